# baseline (speedup 1.0000x reference)
import jax
import jax.numpy as jnp
from jax.experimental import pallas as pl
from jax.experimental.pallas import tpu as pltpu

BEV_H = 200
BEV_W = 176
HW = BEV_H * BEV_W
IN_CH = 128
OUT_CH = 256
B = 4


def _conv_bn_relu_body(x_ref, w_ref, s_ref, b_ref, o_ref):
    x = x_ref[0]
    w = w_ref[...]
    acc = jax.lax.dot_general(w, x, (((1,), (1,)), ((), ())),
                              preferred_element_type=jnp.float32)
    o_ref[0] = jnp.maximum(acc * s_ref[...] + b_ref[...], 0.0)


def _conv_bn_relu(bev3, W, scale2, beta2):
    TS = 3200
    nt = HW // TS
    return pl.pallas_call(
        _conv_bn_relu_body,
        grid=(B, nt),
        in_specs=[
            pl.BlockSpec((1, TS, IN_CH), lambda b, t: (b, t, 0)),
            pl.BlockSpec((OUT_CH, IN_CH), lambda b, t: (0, 0)),
            pl.BlockSpec((OUT_CH, 1), lambda b, t: (0, 0)),
            pl.BlockSpec((OUT_CH, 1), lambda b, t: (0, 0)),
        ],
        out_specs=pl.BlockSpec((1, OUT_CH, TS), lambda b, t: (b, 0, t)),
        out_shape=jax.ShapeDtypeStruct((B, OUT_CH, HW), jnp.float32),
    )(bev3, W, scale2, beta2)


def kernel(features, coordinates, batch_size, W, gamma, beta):
    bidx = coordinates[:, 0]
    y = coordinates[:, 2]
    x = coordinates[:, 3]
    flat = bidx * HW + y * BEV_W + x
    num_seg = B * HW
    seg_max = jax.ops.segment_max(features, flat, num_segments=num_seg)
    counts = jnp.zeros((num_seg,), dtype=jnp.int32).at[flat].add(1)
    bev = jnp.where(counts[:, None] > 0, seg_max, 0.0)
    bev3 = bev.reshape(B, HW, IN_CH)
    scale2 = (gamma / jnp.sqrt(1.0 + 1e-5)).reshape(OUT_CH, 1)
    beta2 = beta.reshape(OUT_CH, 1)
    out = _conv_bn_relu(bev3, W, scale2, beta2)
    return out.reshape(B, OUT_CH, BEV_H, BEV_W)
